# baseline (speedup 1.0000x reference)
import functools
import jax
import jax.numpy as jnp
from jax import lax
from jax.experimental import pallas as pl
from jax.experimental.pallas import tpu as pltpu
from jax.experimental.pallas import tpu_sc as plsc

_N = 10000
_E = 320000
_D = 128
_C = 64

_NSC = 2
_NSUB = 16
_NW = _NSC * _NSUB

_N_PAD = 10240
_ROWS_TILE = _N_PAD // _NSUB
_CHUNK = 128
_CH_PER_TILE = 80
_E_TILE = _CH_PER_TILE * _CHUNK
_E_PAD = _NW * _E_TILE
_CH_A = 80
_CH_B = 160 - _CH_A
_CH_MAX = max(_CH_A, _CH_B)

_mesh = plsc.VectorSubcoreMesh(core_axis_name="c", subcore_axis_name="s")


_DEG_W = 16


def _deg_body(dst_hbm, ones_hbm, zeros_hbm, out_hbm, dst_v, ones_v, acc_sh):
    c = lax.axis_index("c")
    s = lax.axis_index("s")
    w = c * _NSUB + s

    pltpu.sync_copy(zeros_hbm.at[pl.ds(s * _ROWS_TILE, _ROWS_TILE)],
                    acc_sh.at[pl.ds(s * _ROWS_TILE, _ROWS_TILE)])
    pltpu.sync_copy(dst_hbm.at[w], dst_v)
    pltpu.sync_copy(ones_hbm, ones_v)
    plsc.subcore_barrier()

    def body(j, _):
        pltpu.sync_copy(ones_v, acc_sh.at[dst_v.at[j]], add=True)
        return 0

    @pl.when(c == 0)
    def _():
        lax.fori_loop(0, _CH_A, body, 0)

    @pl.when(c == 1)
    def _():
        lax.fori_loop(0, _CH_B, body, 0)

    plsc.subcore_barrier()
    pltpu.sync_copy(acc_sh.at[pl.ds(s * _ROWS_TILE, _ROWS_TILE)],
                    out_hbm.at[c, pl.ds(s * _ROWS_TILE, _ROWS_TILE)])


@jax.jit
def _deg_call(dstp, ones_blk, zeros_deg):
    k = functools.partial(
        pl.kernel,
        mesh=_mesh,
        compiler_params=pltpu.CompilerParams(use_tc_tiling_on_sc=False),
        out_type=jax.ShapeDtypeStruct((_NSC, _N_PAD, _DEG_W), jnp.float32),
        scratch_types=[
            pltpu.VMEM((_CH_MAX, _CHUNK), jnp.int32),
            pltpu.VMEM((_CHUNK, _DEG_W), jnp.float32),
            pltpu.VMEM_SHARED((_N_PAD, _DEG_W), jnp.float32),
        ],
    )(_deg_body)
    return k(dstp, ones_blk, zeros_deg)


def _hop_body(y_hbm, src_hbm, dst_hbm, zeros_hbm, out_hbm,
              src_v, dst_v, rows_v, acc_sh, sem0, sem1):
    c = lax.axis_index("c")
    s = lax.axis_index("s")
    w = c * _NSUB + s
    sems = (sem0, sem1)

    pltpu.sync_copy(zeros_hbm.at[pl.ds(s * _ROWS_TILE, _ROWS_TILE)],
                    acc_sh.at[pl.ds(s * _ROWS_TILE, _ROWS_TILE)])
    pltpu.sync_copy(src_hbm.at[w], src_v)
    pltpu.sync_copy(dst_hbm.at[w], dst_v)
    plsc.subcore_barrier()

    def body(j, _):
        pltpu.async_copy(y_hbm.at[src_v.at[j]], rows_v, sems[0]).wait()
        pltpu.sync_copy(rows_v, acc_sh.at[dst_v.at[j]], add=True)
        return 0

    lax.fori_loop(0, _CH_MAX, body, 0)

    plsc.subcore_barrier()
    pltpu.sync_copy(acc_sh.at[pl.ds(s * _ROWS_TILE, _ROWS_TILE)],
                    out_hbm.at[c, pl.ds(s * _ROWS_TILE, _ROWS_TILE)])


@jax.jit
def _hop_call(y, srcp, dstp, zeros_pad):
    k = functools.partial(
        pl.kernel,
        mesh=_mesh,
        compiler_params=pltpu.CompilerParams(use_tc_tiling_on_sc=False),
        out_type=jax.ShapeDtypeStruct((_NSC, _N_PAD, _C), jnp.float32),
        scratch_types=[
            pltpu.VMEM((_CH_MAX, _CHUNK), jnp.int32),
            pltpu.VMEM((_CH_MAX, _CHUNK), jnp.int32),
            pltpu.VMEM((_CHUNK, _C), jnp.float32),
            pltpu.VMEM_SHARED((_N_PAD, _C), jnp.float32),
            pltpu.SemaphoreType.DMA,
            pltpu.SemaphoreType.DMA,
        ],
    )(_hop_body)
    return k(y, srcp, dstp, zeros_pad)


_BLK = 256


def _prep_body(feat_ref, w_ref, degp_ref, y0_ref):
    deg = degp_ref[0, :, 0] + degp_ref[1, :, 0]
    norm = lax.rsqrt(jnp.maximum(deg, 1.0))
    acc = jnp.dot(feat_ref[...], w_ref[...],
                  preferred_element_type=jnp.float32)
    y0_ref[...] = acc * norm[:, None]


@jax.jit
def _prep_call(featp, W, degp):
    return pl.pallas_call(
        _prep_body,
        grid=(_N_PAD // _BLK,),
        in_specs=[
            pl.BlockSpec((_BLK, _D), lambda i: (i, 0)),
            pl.BlockSpec((_D, _C), lambda i: (0, 0)),
            pl.BlockSpec((_NSC, _BLK, _DEG_W), lambda i: (0, i, 0)),
        ],
        out_specs=pl.BlockSpec((_BLK, _C), lambda i: (i, 0)),
        out_shape=jax.ShapeDtypeStruct((_N_PAD, _C), jnp.float32),
    )(featp, W, degp)


def _mid_body(p_ref, degp_ref, y_ref):
    deg = jnp.maximum(degp_ref[0, :, 0] + degp_ref[1, :, 0], 1.0)
    y_ref[...] = (p_ref[0] + p_ref[1]) * (1.0 / deg)[:, None]


@jax.jit
def _mid_call(p, degp):
    return pl.pallas_call(
        _mid_body,
        grid=(_N_PAD // _BLK,),
        in_specs=[
            pl.BlockSpec((_NSC, _BLK, _C), lambda i: (0, i, 0)),
            pl.BlockSpec((_NSC, _BLK, _DEG_W), lambda i: (0, i, 0)),
        ],
        out_specs=pl.BlockSpec((_BLK, _C), lambda i: (i, 0)),
        out_shape=jax.ShapeDtypeStruct((_N_PAD, _C), jnp.float32),
    )(p, degp)


def _fin_body(p_ref, degp_ref, out_ref, logits_ref):
    deg = jnp.maximum(degp_ref[0, :, 0] + degp_ref[1, :, 0], 1.0)
    norm = lax.rsqrt(deg)
    logits = (p_ref[0] + p_ref[1]) * norm[:, None]
    logits_ref[...] = logits
    m = jnp.max(logits, axis=1, keepdims=True)
    e = jnp.exp(logits - m)
    out_ref[...] = e / jnp.sum(e, axis=1, keepdims=True)


@jax.jit
def _fin_call(p, degp):
    return pl.pallas_call(
        _fin_body,
        grid=(_N_PAD // _BLK,),
        in_specs=[
            pl.BlockSpec((_NSC, _BLK, _C), lambda i: (0, i, 0)),
            pl.BlockSpec((_NSC, _BLK, _DEG_W), lambda i: (0, i, 0)),
        ],
        out_specs=[
            pl.BlockSpec((_BLK, _C), lambda i: (i, 0)),
            pl.BlockSpec((_BLK, _C), lambda i: (i, 0)),
        ],
        out_shape=[
            jax.ShapeDtypeStruct((_N_PAD, _C), jnp.float32),
            jax.ShapeDtypeStruct((_N_PAD, _C), jnp.float32),
        ],
    )(p, degp)


def kernel(features, edge_index, W):
    src = edge_index[0]
    dst = edge_index[1]
    pad_idx = jnp.full((_E_PAD - _E,), _N_PAD - 1, jnp.int32)
    ea = _NSUB * _CH_A * _CHUNK

    def pack(x):
        xp = jnp.concatenate([x, pad_idx])
        xa = xp[:ea].reshape(_NSUB, _CH_A, _CHUNK)
        xb = xp[ea:].reshape(_NSUB, _CH_B, _CHUNK)
        xa = jnp.pad(xa, ((0, 0), (0, _CH_MAX - _CH_A), (0, 0)),
                     constant_values=_N_PAD - 1)
        xb = jnp.pad(xb, ((0, 0), (0, _CH_MAX - _CH_B), (0, 0)),
                     constant_values=_N_PAD - 1)
        return jnp.concatenate([xa, xb], axis=0)

    srcp = pack(src)
    dstp = pack(dst)
    featp = jnp.pad(features, ((0, _N_PAD - _N), (0, 0)))
    zeros_pad = jnp.zeros((_N_PAD, _C), jnp.float32)

    ones_blk = jnp.ones((_CHUNK, _DEG_W), jnp.float32)
    zeros_deg = jnp.zeros((_N_PAD, _DEG_W), jnp.float32)
    degp = _deg_call(dstp, ones_blk, zeros_deg)
    y0 = _prep_call(featp, W, degp)
    p1 = _hop_call(y0, srcp, dstp, zeros_pad)
    y1 = _mid_call(p1, degp)
    p2 = _hop_call(y1, srcp, dstp, zeros_pad)
    out_pad, logits_pad = _fin_call(p2, degp)
    return out_pad[:_N], logits_pad[:_N]


# speedup vs baseline: 1.0777x; 1.0777x over previous
import functools
import jax
import jax.numpy as jnp
from jax import lax
from jax.experimental import pallas as pl
from jax.experimental.pallas import tpu as pltpu
from jax.experimental.pallas import tpu_sc as plsc

_N = 10000
_E = 320000
_D = 128
_C = 64

_NSC = 2
_NSUB = 16
_NW = _NSC * _NSUB

_N_PAD = 10240
_ROWS_TILE = _N_PAD // _NSUB
_CHUNK = 128
_CH_PER_TILE = 80
_E_TILE = _CH_PER_TILE * _CHUNK
_E_PAD = _NW * _E_TILE
_CH_A = 80
_CH_B = 160 - _CH_A
_CH_MAX = max(_CH_A, _CH_B)

_mesh = plsc.VectorSubcoreMesh(core_axis_name="c", subcore_axis_name="s")


_DEG_W = 16


def _deg_body(dst_hbm, ones_hbm, zeros_hbm, out_hbm, dst_v, ones_v, acc_sh):
    c = lax.axis_index("c")
    s = lax.axis_index("s")
    w = c * _NSUB + s

    pltpu.sync_copy(zeros_hbm.at[pl.ds(s * _ROWS_TILE, _ROWS_TILE)],
                    acc_sh.at[pl.ds(s * _ROWS_TILE, _ROWS_TILE)])
    off = jnp.where(c == 0, s * _CH_A, _NSUB * _CH_A + s * _CH_B)
    nch = jnp.where(c == 0, _CH_A, _CH_B)
    pltpu.sync_copy(dst_hbm.at[pl.ds(off, nch)], dst_v.at[pl.ds(0, nch)])
    pltpu.sync_copy(ones_hbm, ones_v)
    plsc.subcore_barrier()

    def body(j, _):
        pltpu.sync_copy(ones_v, acc_sh.at[dst_v.at[j]], add=True)
        return 0

    @pl.when(c == 0)
    def _():
        lax.fori_loop(0, _CH_A, body, 0)

    @pl.when(c == 1)
    def _():
        lax.fori_loop(0, _CH_B, body, 0)

    plsc.subcore_barrier()
    pltpu.sync_copy(acc_sh.at[pl.ds(s * _ROWS_TILE, _ROWS_TILE)],
                    out_hbm.at[c, pl.ds(s * _ROWS_TILE, _ROWS_TILE)])


@jax.jit
def _deg_call(dstp, ones_blk, zeros_deg):
    k = functools.partial(
        pl.kernel,
        mesh=_mesh,
        compiler_params=pltpu.CompilerParams(use_tc_tiling_on_sc=False),
        out_type=jax.ShapeDtypeStruct((_NSC, _N_PAD, _DEG_W), jnp.float32),
        scratch_types=[
            pltpu.VMEM((_CH_MAX, _CHUNK), jnp.int32),
            pltpu.VMEM((_CHUNK, _DEG_W), jnp.float32),
            pltpu.VMEM_SHARED((_N_PAD, _DEG_W), jnp.float32),
        ],
    )(_deg_body)
    return k(dstp, ones_blk, zeros_deg)


def _hop_body(y_hbm, src_hbm, dst_hbm, zeros_hbm, out_hbm,
              src_v, dst_v, rows_v, acc_sh, sem0, sem1):
    c = lax.axis_index("c")
    s = lax.axis_index("s")
    w = c * _NSUB + s
    sems = (sem0, sem1)

    pltpu.sync_copy(zeros_hbm.at[pl.ds(s * _ROWS_TILE, _ROWS_TILE)],
                    acc_sh.at[pl.ds(s * _ROWS_TILE, _ROWS_TILE)])
    off = jnp.where(c == 0, s * _CH_A, _NSUB * _CH_A + s * _CH_B)
    nch = jnp.where(c == 0, _CH_A, _CH_B)
    pltpu.sync_copy(src_hbm.at[pl.ds(off, nch)], src_v.at[pl.ds(0, nch)])
    pltpu.sync_copy(dst_hbm.at[pl.ds(off, nch)], dst_v.at[pl.ds(0, nch)])
    plsc.subcore_barrier()

    def body(j, _):
        pltpu.async_copy(y_hbm.at[src_v.at[j]], rows_v, sems[0]).wait()
        pltpu.sync_copy(rows_v, acc_sh.at[dst_v.at[j]], add=True)
        return 0

    @pl.when(c == 0)
    def _():
        lax.fori_loop(0, _CH_A, body, 0)

    @pl.when(c == 1)
    def _():
        lax.fori_loop(0, _CH_B, body, 0)

    plsc.subcore_barrier()
    pltpu.sync_copy(acc_sh.at[pl.ds(s * _ROWS_TILE, _ROWS_TILE)],
                    out_hbm.at[c, pl.ds(s * _ROWS_TILE, _ROWS_TILE)])


@jax.jit
def _hop_call(y, srcp, dstp, zeros_pad):
    k = functools.partial(
        pl.kernel,
        mesh=_mesh,
        compiler_params=pltpu.CompilerParams(use_tc_tiling_on_sc=False),
        out_type=jax.ShapeDtypeStruct((_NSC, _N_PAD, _C), jnp.float32),
        scratch_types=[
            pltpu.VMEM((_CH_MAX, _CHUNK), jnp.int32),
            pltpu.VMEM((_CH_MAX, _CHUNK), jnp.int32),
            pltpu.VMEM((_CHUNK, _C), jnp.float32),
            pltpu.VMEM_SHARED((_N_PAD, _C), jnp.float32),
            pltpu.SemaphoreType.DMA,
            pltpu.SemaphoreType.DMA,
        ],
    )(_hop_body)
    return k(y, srcp, dstp, zeros_pad)


_BLK = 256


def _prep_body(feat_ref, w_ref, degp_ref, y0_ref):
    deg = degp_ref[0, :, 0] + degp_ref[1, :, 0]
    norm = lax.rsqrt(jnp.maximum(deg, 1.0))
    acc = jnp.dot(feat_ref[...], w_ref[...],
                  preferred_element_type=jnp.float32)
    y0_ref[...] = acc * norm[:, None]


@jax.jit
def _prep_call(featp, W, degp):
    return pl.pallas_call(
        _prep_body,
        grid=(_N_PAD // _BLK,),
        in_specs=[
            pl.BlockSpec((_BLK, _D), lambda i: (i, 0)),
            pl.BlockSpec((_D, _C), lambda i: (0, 0)),
            pl.BlockSpec((_NSC, _BLK, _DEG_W), lambda i: (0, i, 0)),
        ],
        out_specs=pl.BlockSpec((_BLK, _C), lambda i: (i, 0)),
        out_shape=jax.ShapeDtypeStruct((_N_PAD, _C), jnp.float32),
    )(featp, W, degp)


def _mid_body(p_ref, degp_ref, y_ref):
    deg = jnp.maximum(degp_ref[0, :, 0] + degp_ref[1, :, 0], 1.0)
    y_ref[...] = (p_ref[0] + p_ref[1]) * (1.0 / deg)[:, None]


@jax.jit
def _mid_call(p, degp):
    return pl.pallas_call(
        _mid_body,
        grid=(_N_PAD // _BLK,),
        in_specs=[
            pl.BlockSpec((_NSC, _BLK, _C), lambda i: (0, i, 0)),
            pl.BlockSpec((_NSC, _BLK, _DEG_W), lambda i: (0, i, 0)),
        ],
        out_specs=pl.BlockSpec((_BLK, _C), lambda i: (i, 0)),
        out_shape=jax.ShapeDtypeStruct((_N_PAD, _C), jnp.float32),
    )(p, degp)


def _fin_body(p_ref, degp_ref, out_ref, logits_ref):
    deg = jnp.maximum(degp_ref[0, :, 0] + degp_ref[1, :, 0], 1.0)
    norm = lax.rsqrt(deg)
    logits = (p_ref[0] + p_ref[1]) * norm[:, None]
    logits_ref[...] = logits
    m = jnp.max(logits, axis=1, keepdims=True)
    e = jnp.exp(logits - m)
    out_ref[...] = e / jnp.sum(e, axis=1, keepdims=True)


@jax.jit
def _fin_call(p, degp):
    return pl.pallas_call(
        _fin_body,
        grid=(_N_PAD // _BLK,),
        in_specs=[
            pl.BlockSpec((_NSC, _BLK, _C), lambda i: (0, i, 0)),
            pl.BlockSpec((_NSC, _BLK, _DEG_W), lambda i: (0, i, 0)),
        ],
        out_specs=[
            pl.BlockSpec((_BLK, _C), lambda i: (i, 0)),
            pl.BlockSpec((_BLK, _C), lambda i: (i, 0)),
        ],
        out_shape=[
            jax.ShapeDtypeStruct((_N_PAD, _C), jnp.float32),
            jax.ShapeDtypeStruct((_N_PAD, _C), jnp.float32),
        ],
    )(p, degp)


def kernel(features, edge_index, W):
    src = edge_index[0]
    dst = edge_index[1]
    pad_idx = jnp.full((_E_PAD - _E,), _N_PAD - 1, jnp.int32)
    srcp = jnp.concatenate([src, pad_idx]).reshape(_E_PAD // _CHUNK, _CHUNK)
    dstp = jnp.concatenate([dst, pad_idx]).reshape(_E_PAD // _CHUNK, _CHUNK)
    featp = jnp.pad(features, ((0, _N_PAD - _N), (0, 0)))
    zeros_pad = jnp.zeros((_N_PAD, _C), jnp.float32)

    ones_blk = jnp.ones((_CHUNK, _DEG_W), jnp.float32)
    zeros_deg = jnp.zeros((_N_PAD, _DEG_W), jnp.float32)
    degp = _deg_call(dstp, ones_blk, zeros_deg)
    y0 = _prep_call(featp, W, degp)
    p1 = _hop_call(y0, srcp, dstp, zeros_pad)
    y1 = _mid_call(p1, degp)
    p2 = _hop_call(y1, srcp, dstp, zeros_pad)
    out_pad, logits_pad = _fin_call(p2, degp)
    return out_pad[:_N], logits_pad[:_N]


# speedup vs baseline: 1.4463x; 1.3421x over previous
import functools
import jax
import jax.numpy as jnp
from jax import lax
from jax.experimental import pallas as pl
from jax.experimental.pallas import tpu as pltpu
from jax.experimental.pallas import tpu_sc as plsc

_N = 10000
_E = 320000
_D = 128
_C = 64

_NSC = 2
_NSUB = 16
_NW = _NSC * _NSUB

_N_PAD = 10240
_ROWS_TILE = _N_PAD // _NSUB
_CHUNK = 128
_CH_PER_TILE = -(-_E // (_NW * _CHUNK))
_E_TILE = _CH_PER_TILE * _CHUNK
_E_PAD = _NW * _E_TILE

_mesh = plsc.VectorSubcoreMesh(core_axis_name="c", subcore_axis_name="s")


_DEG_W = 16


def _deg_body(dst_hbm, ones_hbm, zeros_hbm, out_hbm, dst_v, ones_v, acc_sh):
    c = lax.axis_index("c")
    s = lax.axis_index("s")
    w = c * _NSUB + s

    pltpu.sync_copy(zeros_hbm.at[pl.ds(s * _ROWS_TILE, _ROWS_TILE)],
                    acc_sh.at[pl.ds(s * _ROWS_TILE, _ROWS_TILE)])
    pltpu.sync_copy(dst_hbm.at[w], dst_v)
    pltpu.sync_copy(ones_hbm, ones_v)
    plsc.subcore_barrier()

    def body(j, _):
        pltpu.sync_copy(ones_v, acc_sh.at[dst_v.at[j]], add=True)
        return 0

    lax.fori_loop(0, _CH_PER_TILE, body, 0)

    plsc.subcore_barrier()
    pltpu.sync_copy(acc_sh.at[pl.ds(s * _ROWS_TILE, _ROWS_TILE)],
                    out_hbm.at[c, pl.ds(s * _ROWS_TILE, _ROWS_TILE)])


@jax.jit
def _deg_call(dstp, ones_blk, zeros_deg):
    k = functools.partial(
        pl.kernel,
        mesh=_mesh,
        compiler_params=pltpu.CompilerParams(use_tc_tiling_on_sc=False),
        out_type=jax.ShapeDtypeStruct((_NSC, _N_PAD, _DEG_W), jnp.float32),
        scratch_types=[
            pltpu.VMEM((_CH_PER_TILE, _CHUNK), jnp.int32),
            pltpu.VMEM((_CHUNK, _DEG_W), jnp.float32),
            pltpu.VMEM_SHARED((_N_PAD, _DEG_W), jnp.float32),
        ],
    )(_deg_body)
    return k(dstp, ones_blk, zeros_deg)


def _hop_body(y_hbm, src_hbm, dst_hbm, zeros_hbm, out_hbm,
              src_v, dst_v, rows_v, acc_sh, sem):
    c = lax.axis_index("c")
    s = lax.axis_index("s")
    w = c * _NSUB + s

    pltpu.sync_copy(zeros_hbm.at[pl.ds(s * _ROWS_TILE, _ROWS_TILE)],
                    acc_sh.at[pl.ds(s * _ROWS_TILE, _ROWS_TILE)])
    pltpu.sync_copy(src_hbm.at[w], src_v)
    pltpu.sync_copy(dst_hbm.at[w], dst_v)
    plsc.subcore_barrier()

    def body(j, _):
        pltpu.async_copy(y_hbm.at[src_v.at[j]], rows_v, sem).wait()
        pltpu.sync_copy(rows_v, acc_sh.at[dst_v.at[j]], add=True)
        return 0

    lax.fori_loop(0, _CH_PER_TILE, body, 0)

    plsc.subcore_barrier()
    pltpu.sync_copy(acc_sh.at[pl.ds(s * _ROWS_TILE, _ROWS_TILE)],
                    out_hbm.at[c, pl.ds(s * _ROWS_TILE, _ROWS_TILE)])


@jax.jit
def _hop_call(y, srcp, dstp, zeros_pad):
    k = functools.partial(
        pl.kernel,
        mesh=_mesh,
        compiler_params=pltpu.CompilerParams(use_tc_tiling_on_sc=False),
        out_type=jax.ShapeDtypeStruct((_NSC, _N_PAD, _C), jnp.float32),
        scratch_types=[
            pltpu.VMEM((_CH_PER_TILE, _CHUNK), jnp.int32),
            pltpu.VMEM((_CH_PER_TILE, _CHUNK), jnp.int32),
            pltpu.VMEM((_CHUNK, _C), jnp.float32),
            pltpu.VMEM_SHARED((_N_PAD, _C), jnp.float32),
            pltpu.SemaphoreType.DMA,
        ],
    )(_hop_body)
    return k(y, srcp, dstp, zeros_pad)


_BLK = 256


def _prep_body(feat_ref, w_ref, degp_ref, y0_ref):
    deg = degp_ref[0, :, 0] + degp_ref[1, :, 0]
    norm = lax.rsqrt(jnp.maximum(deg, 1.0))
    acc = jnp.dot(feat_ref[...], w_ref[...],
                  preferred_element_type=jnp.float32)
    y0_ref[...] = acc * norm[:, None]


@jax.jit
def _prep_call(featp, W, degp):
    return pl.pallas_call(
        _prep_body,
        grid=(_N_PAD // _BLK,),
        in_specs=[
            pl.BlockSpec((_BLK, _D), lambda i: (i, 0)),
            pl.BlockSpec((_D, _C), lambda i: (0, 0)),
            pl.BlockSpec((_NSC, _BLK, _DEG_W), lambda i: (0, i, 0)),
        ],
        out_specs=pl.BlockSpec((_BLK, _C), lambda i: (i, 0)),
        out_shape=jax.ShapeDtypeStruct((_N_PAD, _C), jnp.float32),
    )(featp, W, degp)


def _mid_body(p_ref, degp_ref, y_ref):
    deg = jnp.maximum(degp_ref[0, :, 0] + degp_ref[1, :, 0], 1.0)
    y_ref[...] = (p_ref[0] + p_ref[1]) * (1.0 / deg)[:, None]


@jax.jit
def _mid_call(p, degp):
    return pl.pallas_call(
        _mid_body,
        grid=(_N_PAD // _BLK,),
        in_specs=[
            pl.BlockSpec((_NSC, _BLK, _C), lambda i: (0, i, 0)),
            pl.BlockSpec((_NSC, _BLK, _DEG_W), lambda i: (0, i, 0)),
        ],
        out_specs=pl.BlockSpec((_BLK, _C), lambda i: (i, 0)),
        out_shape=jax.ShapeDtypeStruct((_N_PAD, _C), jnp.float32),
    )(p, degp)


def _fin_body(p_ref, degp_ref, out_ref, logits_ref):
    deg = jnp.maximum(degp_ref[0, :, 0] + degp_ref[1, :, 0], 1.0)
    norm = lax.rsqrt(deg)
    logits = (p_ref[0] + p_ref[1]) * norm[:, None]
    logits_ref[...] = logits
    m = jnp.max(logits, axis=1, keepdims=True)
    e = jnp.exp(logits - m)
    out_ref[...] = e / jnp.sum(e, axis=1, keepdims=True)


@jax.jit
def _fin_call(p, degp):
    return pl.pallas_call(
        _fin_body,
        grid=(_N_PAD // _BLK,),
        in_specs=[
            pl.BlockSpec((_NSC, _BLK, _C), lambda i: (0, i, 0)),
            pl.BlockSpec((_NSC, _BLK, _DEG_W), lambda i: (0, i, 0)),
        ],
        out_specs=[
            pl.BlockSpec((_BLK, _C), lambda i: (i, 0)),
            pl.BlockSpec((_BLK, _C), lambda i: (i, 0)),
        ],
        out_shape=[
            jax.ShapeDtypeStruct((_N_PAD, _C), jnp.float32),
            jax.ShapeDtypeStruct((_N_PAD, _C), jnp.float32),
        ],
    )(p, degp)


def kernel(features, edge_index, W):
    src = edge_index[0]
    dst = edge_index[1]
    pad_idx = jnp.full((_E_PAD - _E,), _N_PAD - 1, jnp.int32)
    srcp = jnp.concatenate([src, pad_idx]).reshape(_NW, _CH_PER_TILE, _CHUNK)
    dstp = jnp.concatenate([dst, pad_idx]).reshape(_NW, _CH_PER_TILE, _CHUNK)
    featp = jnp.pad(features, ((0, _N_PAD - _N), (0, 0)))
    zeros_pad = jnp.zeros((_N_PAD, _C), jnp.float32)
    ones_blk = jnp.ones((_CHUNK, _DEG_W), jnp.float32)
    zeros_deg = jnp.zeros((_N_PAD, _DEG_W), jnp.float32)

    degp = _deg_call(dstp, ones_blk, zeros_deg)
    y0 = _prep_call(featp, W, degp)
    p1 = _hop_call(y0, srcp, dstp, zeros_pad)
    y1 = _mid_call(p1, degp)
    p2 = _hop_call(y1, srcp, dstp, zeros_pad)
    out_pad, logits_pad = _fin_call(p2, degp)
    return out_pad[:_N], logits_pad[:_N]


# speedup vs baseline: 1.9685x; 1.3611x over previous
import functools
import jax
import jax.numpy as jnp
from jax import lax
from jax.experimental import pallas as pl
from jax.experimental.pallas import tpu as pltpu
from jax.experimental.pallas import tpu_sc as plsc

_N = 10000
_E = 320000
_D = 128
_C = 64

_NSC = 2
_NSUB = 16
_NW = _NSC * _NSUB

_N_PAD = 10240
_ROWS_TILE = _N_PAD // _NSUB
_CHUNK = 128
_CH_PER_TILE = -(-_E // (_NW * _CHUNK))
_E_TILE = _CH_PER_TILE * _CHUNK
_E_PAD = _NW * _E_TILE

_mesh = plsc.VectorSubcoreMesh(core_axis_name="c", subcore_axis_name="s")


_DEG_W = 16


def _deg_body(dst_hbm, ones_hbm, zeros_hbm, out_hbm, dst_v, ones_v, acc_sh):
    c = lax.axis_index("c")
    s = lax.axis_index("s")
    w = c * _NSUB + s

    pltpu.sync_copy(zeros_hbm.at[pl.ds(s * _ROWS_TILE, _ROWS_TILE)],
                    acc_sh.at[pl.ds(s * _ROWS_TILE, _ROWS_TILE)])
    pltpu.sync_copy(dst_hbm.at[w], dst_v)
    pltpu.sync_copy(ones_hbm, ones_v)
    plsc.subcore_barrier()

    def body(j, _):
        pltpu.sync_copy(ones_v, acc_sh.at[dst_v.at[j]], add=True)
        return 0

    lax.fori_loop(0, _CH_PER_TILE, body, 0)

    plsc.subcore_barrier()
    pltpu.sync_copy(acc_sh.at[pl.ds(s * _ROWS_TILE, _ROWS_TILE)],
                    out_hbm.at[c, pl.ds(s * _ROWS_TILE, _ROWS_TILE)])


@jax.jit
def _deg_call(dstp, ones_blk, zeros_deg):
    k = functools.partial(
        pl.kernel,
        mesh=_mesh,
        compiler_params=pltpu.CompilerParams(use_tc_tiling_on_sc=False),
        out_type=jax.ShapeDtypeStruct((_NSC, _N_PAD, _DEG_W), jnp.float32),
        scratch_types=[
            pltpu.VMEM((_CH_PER_TILE, _CHUNK), jnp.int32),
            pltpu.VMEM((_CHUNK, _DEG_W), jnp.float32),
            pltpu.VMEM_SHARED((_N_PAD, _DEG_W), jnp.float32),
        ],
    )(_deg_body)
    return k(dstp, ones_blk, zeros_deg)


def _hop_body(y_hbm, src_hbm, dst_hbm, zeros_hbm, out_hbm,
              src_v, dst_v, rows_v, acc_sh, y_sh, sem):
    c = lax.axis_index("c")
    s = lax.axis_index("s")
    w = c * _NSUB + s

    pltpu.sync_copy(zeros_hbm.at[pl.ds(s * _ROWS_TILE, _ROWS_TILE)],
                    acc_sh.at[pl.ds(s * _ROWS_TILE, _ROWS_TILE)])
    pltpu.sync_copy(y_hbm.at[pl.ds(s * _ROWS_TILE, _ROWS_TILE)],
                    y_sh.at[pl.ds(s * _ROWS_TILE, _ROWS_TILE)])
    pltpu.sync_copy(src_hbm.at[w], src_v)
    pltpu.sync_copy(dst_hbm.at[w], dst_v)
    plsc.subcore_barrier()

    def body(j, _):
        pltpu.async_copy(y_sh.at[src_v.at[j]], rows_v, sem).wait()
        pltpu.sync_copy(rows_v, acc_sh.at[dst_v.at[j]], add=True)
        return 0

    lax.fori_loop(0, _CH_PER_TILE, body, 0)

    plsc.subcore_barrier()
    pltpu.sync_copy(acc_sh.at[pl.ds(s * _ROWS_TILE, _ROWS_TILE)],
                    out_hbm.at[c, pl.ds(s * _ROWS_TILE, _ROWS_TILE)])


@jax.jit
def _hop_call(y, srcp, dstp, zeros_pad):
    k = functools.partial(
        pl.kernel,
        mesh=_mesh,
        compiler_params=pltpu.CompilerParams(use_tc_tiling_on_sc=False),
        out_type=jax.ShapeDtypeStruct((_NSC, _N_PAD, _C), jnp.float32),
        scratch_types=[
            pltpu.VMEM((_CH_PER_TILE, _CHUNK), jnp.int32),
            pltpu.VMEM((_CH_PER_TILE, _CHUNK), jnp.int32),
            pltpu.VMEM((_CHUNK, _C), jnp.float32),
            pltpu.VMEM_SHARED((_N_PAD, _C), jnp.float32),
            pltpu.VMEM_SHARED((_N_PAD, _C), jnp.float32),
            pltpu.SemaphoreType.DMA,
        ],
    )(_hop_body)
    return k(y, srcp, dstp, zeros_pad)


_BLK = 256


def _prep_body(feat_ref, w_ref, degp_ref, y0_ref):
    deg = degp_ref[0, :, 0] + degp_ref[1, :, 0]
    norm = lax.rsqrt(jnp.maximum(deg, 1.0))
    acc = jnp.dot(feat_ref[...], w_ref[...],
                  preferred_element_type=jnp.float32)
    y0_ref[...] = acc * norm[:, None]


@jax.jit
def _prep_call(featp, W, degp):
    return pl.pallas_call(
        _prep_body,
        grid=(_N_PAD // _BLK,),
        in_specs=[
            pl.BlockSpec((_BLK, _D), lambda i: (i, 0)),
            pl.BlockSpec((_D, _C), lambda i: (0, 0)),
            pl.BlockSpec((_NSC, _BLK, _DEG_W), lambda i: (0, i, 0)),
        ],
        out_specs=pl.BlockSpec((_BLK, _C), lambda i: (i, 0)),
        out_shape=jax.ShapeDtypeStruct((_N_PAD, _C), jnp.float32),
    )(featp, W, degp)


def _mid_body(p_ref, degp_ref, y_ref):
    deg = jnp.maximum(degp_ref[0, :, 0] + degp_ref[1, :, 0], 1.0)
    y_ref[...] = (p_ref[0] + p_ref[1]) * (1.0 / deg)[:, None]


@jax.jit
def _mid_call(p, degp):
    return pl.pallas_call(
        _mid_body,
        grid=(_N_PAD // _BLK,),
        in_specs=[
            pl.BlockSpec((_NSC, _BLK, _C), lambda i: (0, i, 0)),
            pl.BlockSpec((_NSC, _BLK, _DEG_W), lambda i: (0, i, 0)),
        ],
        out_specs=pl.BlockSpec((_BLK, _C), lambda i: (i, 0)),
        out_shape=jax.ShapeDtypeStruct((_N_PAD, _C), jnp.float32),
    )(p, degp)


def _fin_body(p_ref, degp_ref, out_ref, logits_ref):
    deg = jnp.maximum(degp_ref[0, :, 0] + degp_ref[1, :, 0], 1.0)
    norm = lax.rsqrt(deg)
    logits = (p_ref[0] + p_ref[1]) * norm[:, None]
    logits_ref[...] = logits
    m = jnp.max(logits, axis=1, keepdims=True)
    e = jnp.exp(logits - m)
    out_ref[...] = e / jnp.sum(e, axis=1, keepdims=True)


@jax.jit
def _fin_call(p, degp):
    return pl.pallas_call(
        _fin_body,
        grid=(_N_PAD // _BLK,),
        in_specs=[
            pl.BlockSpec((_NSC, _BLK, _C), lambda i: (0, i, 0)),
            pl.BlockSpec((_NSC, _BLK, _DEG_W), lambda i: (0, i, 0)),
        ],
        out_specs=[
            pl.BlockSpec((_BLK, _C), lambda i: (i, 0)),
            pl.BlockSpec((_BLK, _C), lambda i: (i, 0)),
        ],
        out_shape=[
            jax.ShapeDtypeStruct((_N_PAD, _C), jnp.float32),
            jax.ShapeDtypeStruct((_N_PAD, _C), jnp.float32),
        ],
    )(p, degp)


def kernel(features, edge_index, W):
    src = edge_index[0]
    dst = edge_index[1]
    pad_idx = jnp.full((_E_PAD - _E,), _N_PAD - 1, jnp.int32)
    srcp = jnp.concatenate([src, pad_idx]).reshape(_NW, _CH_PER_TILE, _CHUNK)
    dstp = jnp.concatenate([dst, pad_idx]).reshape(_NW, _CH_PER_TILE, _CHUNK)
    featp = jnp.pad(features, ((0, _N_PAD - _N), (0, 0)))
    zeros_pad = jnp.zeros((_N_PAD, _C), jnp.float32)
    ones_blk = jnp.ones((_CHUNK, _DEG_W), jnp.float32)
    zeros_deg = jnp.zeros((_N_PAD, _DEG_W), jnp.float32)

    degp = _deg_call(dstp, ones_blk, zeros_deg)
    y0 = _prep_call(featp, W, degp)
    p1 = _hop_call(y0, srcp, dstp, zeros_pad)
    y1 = _mid_call(p1, degp)
    p2 = _hop_call(y1, srcp, dstp, zeros_pad)
    out_pad, logits_pad = _fin_call(p2, degp)
    return out_pad[:_N], logits_pad[:_N]


# speedup vs baseline: 2.2996x; 1.1682x over previous
import functools
import jax
import jax.numpy as jnp
from jax import lax
from jax.experimental import pallas as pl
from jax.experimental.pallas import tpu as pltpu
from jax.experimental.pallas import tpu_sc as plsc

_N = 10000
_E = 320000
_D = 128
_C = 64

_NSC = 2
_NSUB = 16
_NW = _NSC * _NSUB

_N_PAD = 10240
_ROWS_TILE = _N_PAD // _NSUB
_CHUNK = 128
_CH_PER_TILE = -(-_E // (_NW * _CHUNK))
_E_TILE = _CH_PER_TILE * _CHUNK
_E_PAD = _NW * _E_TILE

_mesh = plsc.VectorSubcoreMesh(core_axis_name="c", subcore_axis_name="s")


_DEG_W = 16


def _deg_body(dst_hbm, ones_hbm, zeros_hbm, out_hbm, dst_v, ones_v, acc_sh):
    c = lax.axis_index("c")
    s = lax.axis_index("s")
    w = c * _NSUB + s

    pltpu.sync_copy(zeros_hbm.at[pl.ds(s * _ROWS_TILE, _ROWS_TILE)],
                    acc_sh.at[pl.ds(s * _ROWS_TILE, _ROWS_TILE)])
    pltpu.sync_copy(dst_hbm.at[w], dst_v)
    pltpu.sync_copy(ones_hbm, ones_v)
    plsc.subcore_barrier()

    def body(j, _):
        pltpu.sync_copy(ones_v, acc_sh.at[dst_v.at[j]], add=True)
        return 0

    lax.fori_loop(0, _CH_PER_TILE, body, 0)

    plsc.subcore_barrier()
    pltpu.sync_copy(acc_sh.at[pl.ds(s * _ROWS_TILE, _ROWS_TILE)],
                    out_hbm.at[c, pl.ds(s * _ROWS_TILE, _ROWS_TILE)])


@jax.jit
def _deg_call(dstp, ones_blk, zeros_deg):
    k = functools.partial(
        pl.kernel,
        mesh=_mesh,
        compiler_params=pltpu.CompilerParams(use_tc_tiling_on_sc=False),
        out_type=jax.ShapeDtypeStruct((_NSC, _N_PAD, _DEG_W), jnp.float32),
        scratch_types=[
            pltpu.VMEM((_CH_PER_TILE, _CHUNK), jnp.int32),
            pltpu.VMEM((_CHUNK, _DEG_W), jnp.float32),
            pltpu.VMEM_SHARED((_N_PAD, _DEG_W), jnp.float32),
        ],
    )(_deg_body)
    return k(dstp, ones_blk, zeros_deg)


def _hop_body(y_hbm, src_hbm, dst_hbm, zeros_hbm, out_hbm,
              src_v, dst_v, rows_v, acc_sh, y_sh, sem, sem2):
    c = lax.axis_index("c")
    s = lax.axis_index("s")
    w = c * _NSUB + s

    pltpu.sync_copy(zeros_hbm.at[pl.ds(s * _ROWS_TILE, _ROWS_TILE)],
                    acc_sh.at[pl.ds(s * _ROWS_TILE, _ROWS_TILE)])
    pltpu.sync_copy(y_hbm.at[pl.ds(s * _ROWS_TILE, _ROWS_TILE)],
                    y_sh.at[pl.ds(s * _ROWS_TILE, _ROWS_TILE)])
    pltpu.sync_copy(src_hbm.at[w], src_v)
    pltpu.sync_copy(dst_hbm.at[w], dst_v)
    plsc.subcore_barrier()

    pltpu.async_copy(y_sh.at[src_v.at[0]], rows_v.at[0], sem)
    pltpu.async_copy(y_sh.at[src_v.at[1]], rows_v.at[1], sem2)
    sems = (sem, sem2)

    def body(t, _):
        for b in range(2):
            j = 2 * t + b
            pltpu.make_async_copy(y_sh.at[src_v.at[j]],
                                  rows_v.at[b], sems[b]).wait()
            pltpu.sync_copy(rows_v.at[b], acc_sh.at[dst_v.at[j]], add=True)

            @pl.when(j + 2 < _CH_PER_TILE)
            def _():
                pltpu.async_copy(y_sh.at[src_v.at[j + 2]],
                                 rows_v.at[b], sems[b])
        return 0

    lax.fori_loop(0, _CH_PER_TILE // 2, body, 0)

    if _CH_PER_TILE % 2 == 1:
        jl = _CH_PER_TILE - 1
        bl = jl % 2
        pltpu.make_async_copy(y_sh.at[src_v.at[jl]],
                              rows_v.at[bl], sems[bl]).wait()
        pltpu.sync_copy(rows_v.at[bl], acc_sh.at[dst_v.at[jl]], add=True)

    plsc.subcore_barrier()
    pltpu.sync_copy(acc_sh.at[pl.ds(s * _ROWS_TILE, _ROWS_TILE)],
                    out_hbm.at[c, pl.ds(s * _ROWS_TILE, _ROWS_TILE)])


@jax.jit
def _hop_call(y, srcp, dstp, zeros_pad):
    k = functools.partial(
        pl.kernel,
        mesh=_mesh,
        compiler_params=pltpu.CompilerParams(use_tc_tiling_on_sc=False),
        out_type=jax.ShapeDtypeStruct((_NSC, _N_PAD, _C), jnp.float32),
        scratch_types=[
            pltpu.VMEM((_CH_PER_TILE, _CHUNK), jnp.int32),
            pltpu.VMEM((_CH_PER_TILE, _CHUNK), jnp.int32),
            pltpu.VMEM((2, _CHUNK, _C), jnp.float32),
            pltpu.VMEM_SHARED((_N_PAD, _C), jnp.float32),
            pltpu.VMEM_SHARED((_N_PAD, _C), jnp.float32),
            pltpu.SemaphoreType.DMA,
            pltpu.SemaphoreType.DMA,
        ],
    )(_hop_body)
    return k(y, srcp, dstp, zeros_pad)


_BLK = 256


def _prep_body(feat_ref, w_ref, degp_ref, y0_ref):
    deg = degp_ref[0, :, 0] + degp_ref[1, :, 0]
    norm = lax.rsqrt(jnp.maximum(deg, 1.0))
    acc = jnp.dot(feat_ref[...], w_ref[...],
                  preferred_element_type=jnp.float32)
    y0_ref[...] = acc * norm[:, None]


@jax.jit
def _prep_call(featp, W, degp):
    return pl.pallas_call(
        _prep_body,
        grid=(_N_PAD // _BLK,),
        in_specs=[
            pl.BlockSpec((_BLK, _D), lambda i: (i, 0)),
            pl.BlockSpec((_D, _C), lambda i: (0, 0)),
            pl.BlockSpec((_NSC, _BLK, _DEG_W), lambda i: (0, i, 0)),
        ],
        out_specs=pl.BlockSpec((_BLK, _C), lambda i: (i, 0)),
        out_shape=jax.ShapeDtypeStruct((_N_PAD, _C), jnp.float32),
    )(featp, W, degp)


def _mid_body(p_ref, degp_ref, y_ref):
    deg = jnp.maximum(degp_ref[0, :, 0] + degp_ref[1, :, 0], 1.0)
    y_ref[...] = (p_ref[0] + p_ref[1]) * (1.0 / deg)[:, None]


@jax.jit
def _mid_call(p, degp):
    return pl.pallas_call(
        _mid_body,
        grid=(_N_PAD // _BLK,),
        in_specs=[
            pl.BlockSpec((_NSC, _BLK, _C), lambda i: (0, i, 0)),
            pl.BlockSpec((_NSC, _BLK, _DEG_W), lambda i: (0, i, 0)),
        ],
        out_specs=pl.BlockSpec((_BLK, _C), lambda i: (i, 0)),
        out_shape=jax.ShapeDtypeStruct((_N_PAD, _C), jnp.float32),
    )(p, degp)


def _fin_body(p_ref, degp_ref, out_ref, logits_ref):
    deg = jnp.maximum(degp_ref[0, :, 0] + degp_ref[1, :, 0], 1.0)
    norm = lax.rsqrt(deg)
    logits = (p_ref[0] + p_ref[1]) * norm[:, None]
    logits_ref[...] = logits
    m = jnp.max(logits, axis=1, keepdims=True)
    e = jnp.exp(logits - m)
    out_ref[...] = e / jnp.sum(e, axis=1, keepdims=True)


@jax.jit
def _fin_call(p, degp):
    return pl.pallas_call(
        _fin_body,
        grid=(_N_PAD // _BLK,),
        in_specs=[
            pl.BlockSpec((_NSC, _BLK, _C), lambda i: (0, i, 0)),
            pl.BlockSpec((_NSC, _BLK, _DEG_W), lambda i: (0, i, 0)),
        ],
        out_specs=[
            pl.BlockSpec((_BLK, _C), lambda i: (i, 0)),
            pl.BlockSpec((_BLK, _C), lambda i: (i, 0)),
        ],
        out_shape=[
            jax.ShapeDtypeStruct((_N_PAD, _C), jnp.float32),
            jax.ShapeDtypeStruct((_N_PAD, _C), jnp.float32),
        ],
    )(p, degp)


def kernel(features, edge_index, W):
    src = edge_index[0]
    dst = edge_index[1]
    pad_idx = jnp.full((_E_PAD - _E,), _N_PAD - 1, jnp.int32)
    srcp = jnp.concatenate([src, pad_idx]).reshape(_NW, _CH_PER_TILE, _CHUNK)
    dstp = jnp.concatenate([dst, pad_idx]).reshape(_NW, _CH_PER_TILE, _CHUNK)
    featp = jnp.pad(features, ((0, _N_PAD - _N), (0, 0)))
    zeros_pad = jnp.zeros((_N_PAD, _C), jnp.float32)
    ones_blk = jnp.ones((_CHUNK, _DEG_W), jnp.float32)
    zeros_deg = jnp.zeros((_N_PAD, _DEG_W), jnp.float32)

    degp = _deg_call(dstp, ones_blk, zeros_deg)
    y0 = _prep_call(featp, W, degp)
    p1 = _hop_call(y0, srcp, dstp, zeros_pad)
    y1 = _mid_call(p1, degp)
    p2 = _hop_call(y1, srcp, dstp, zeros_pad)
    out_pad, logits_pad = _fin_call(p2, degp)
    return out_pad[:_N], logits_pad[:_N]


# speedup vs baseline: 2.4798x; 1.0783x over previous
import functools
import jax
import jax.numpy as jnp
from jax import lax
from jax.experimental import pallas as pl
from jax.experimental.pallas import tpu as pltpu
from jax.experimental.pallas import tpu_sc as plsc

_N = 10000
_E = 320000
_D = 128
_C = 64

_NSC = 2
_NSUB = 16
_NW = _NSC * _NSUB

_N_PAD = 10240
_ROWS_TILE = _N_PAD // _NSUB
_CHUNK = 128
_CH_PER_TILE = -(-_E // (_NW * _CHUNK))
_E_TILE = _CH_PER_TILE * _CHUNK
_E_PAD = _NW * _E_TILE

_mesh = plsc.VectorSubcoreMesh(core_axis_name="c", subcore_axis_name="s")


_DEG_W = 16


def _deg_body(dst_hbm, ones_hbm, zeros_hbm, out_hbm, dst_v, ones_v, acc_sh):
    c = lax.axis_index("c")
    s = lax.axis_index("s")
    w = c * _NSUB + s

    pltpu.sync_copy(zeros_hbm.at[pl.ds(s * _ROWS_TILE, _ROWS_TILE)],
                    acc_sh.at[pl.ds(s * _ROWS_TILE, _ROWS_TILE)])
    pltpu.sync_copy(dst_hbm.at[w], dst_v)
    pltpu.sync_copy(ones_hbm, ones_v)
    plsc.subcore_barrier()

    def body(j, _):
        pltpu.sync_copy(ones_v, acc_sh.at[dst_v.at[j]], add=True)
        return 0

    lax.fori_loop(0, _CH_PER_TILE, body, 0)

    plsc.subcore_barrier()
    pltpu.sync_copy(acc_sh.at[pl.ds(s * _ROWS_TILE, _ROWS_TILE)],
                    out_hbm.at[c, pl.ds(s * _ROWS_TILE, _ROWS_TILE)])


@jax.jit
def _deg_call(dstp, ones_blk, zeros_deg):
    k = functools.partial(
        pl.kernel,
        mesh=_mesh,
        compiler_params=pltpu.CompilerParams(use_tc_tiling_on_sc=False),
        out_type=jax.ShapeDtypeStruct((_NSC, _N_PAD, _DEG_W), jnp.float32),
        scratch_types=[
            pltpu.VMEM((_CH_PER_TILE, _CHUNK), jnp.int32),
            pltpu.VMEM((_CHUNK, _DEG_W), jnp.float32),
            pltpu.VMEM_SHARED((_N_PAD, _DEG_W), jnp.float32),
        ],
    )(_deg_body)
    return k(dstp, ones_blk, zeros_deg)


def _hop_body(y_hbm, src_hbm, dst_hbm, zeros_hbm, out_hbm,
              src_v, dst_v, rows_v, acc_sh, y_sh, sem, sem2):
    c = lax.axis_index("c")
    s = lax.axis_index("s")
    w = c * _NSUB + s

    pltpu.sync_copy(zeros_hbm.at[pl.ds(s * _ROWS_TILE, _ROWS_TILE)],
                    acc_sh.at[pl.ds(s * _ROWS_TILE, _ROWS_TILE)])
    pltpu.sync_copy(y_hbm.at[pl.ds(s * _ROWS_TILE, _ROWS_TILE)],
                    y_sh.at[pl.ds(s * _ROWS_TILE, _ROWS_TILE)])
    pltpu.sync_copy(src_hbm.at[w], src_v)
    pltpu.sync_copy(dst_hbm.at[w], dst_v)
    plsc.subcore_barrier()

    pltpu.async_copy(y_sh.at[src_v.at[0]], rows_v.at[0], sem)
    pltpu.async_copy(y_sh.at[src_v.at[1]], rows_v.at[1], sem2)
    sems = (sem, sem2)

    def body(t, _):
        for b in range(2):
            j = 2 * t + b
            pltpu.make_async_copy(y_sh.at[src_v.at[j]],
                                  rows_v.at[b], sems[b]).wait()
            pltpu.sync_copy(rows_v.at[b], acc_sh.at[dst_v.at[j]], add=True)

            @pl.when(j + 2 < _CH_PER_TILE)
            def _():
                pltpu.async_copy(y_sh.at[src_v.at[j + 2]],
                                 rows_v.at[b], sems[b])
        return 0

    lax.fori_loop(0, _CH_PER_TILE // 2, body, 0)

    if _CH_PER_TILE % 2 == 1:
        jl = _CH_PER_TILE - 1
        bl = jl % 2
        pltpu.make_async_copy(y_sh.at[src_v.at[jl]],
                              rows_v.at[bl], sems[bl]).wait()
        pltpu.sync_copy(rows_v.at[bl], acc_sh.at[dst_v.at[jl]], add=True)

    plsc.subcore_barrier()
    pltpu.sync_copy(acc_sh.at[pl.ds(s * _ROWS_TILE, _ROWS_TILE)],
                    out_hbm.at[c, pl.ds(s * _ROWS_TILE, _ROWS_TILE)])


@jax.jit
def _hop_call(y, srcp, dstp, zeros_pad):
    k = functools.partial(
        pl.kernel,
        mesh=_mesh,
        compiler_params=pltpu.CompilerParams(use_tc_tiling_on_sc=False),
        out_type=jax.ShapeDtypeStruct((_NSC, _N_PAD, _C), jnp.float32),
        scratch_types=[
            pltpu.VMEM((_CH_PER_TILE, _CHUNK), jnp.int32),
            pltpu.VMEM((_CH_PER_TILE, _CHUNK), jnp.int32),
            pltpu.VMEM((2, _CHUNK, _C), jnp.float32),
            pltpu.VMEM_SHARED((_N_PAD, _C), jnp.float32),
            pltpu.VMEM_SHARED((_N_PAD, _C), jnp.float32),
            pltpu.SemaphoreType.DMA,
            pltpu.SemaphoreType.DMA,
        ],
    )(_hop_body)
    return k(y, srcp, dstp, zeros_pad)


_SB = 320


def _midsc_body(p_hbm, degp_hbm, y_hbm, pa_v, dg_v):
    c = lax.axis_index("c")
    s = lax.axis_index("s")
    w = c * _NSUB + s
    half = _N_PAD // _NW

    r0 = w * half
    pltpu.sync_copy(p_hbm.at[0, pl.ds(r0, _SB)], pa_v.at[0])
    pltpu.sync_copy(p_hbm.at[1, pl.ds(r0, _SB)], pa_v.at[1])
    pltpu.sync_copy(degp_hbm.at[0, pl.ds(r0, _SB)], dg_v.at[0])
    pltpu.sync_copy(degp_hbm.at[1, pl.ds(r0, _SB)], dg_v.at[1])

    def rowfn(i, _):
        d16 = dg_v[0, i] + dg_v[1, i]
        r16 = 1.0 / jnp.maximum(d16, 1.0)
        for q in range(_C // 16):
            v = (pa_v[0, i, pl.ds(16 * q, 16)]
                 + pa_v[1, i, pl.ds(16 * q, 16)]) * r16
            pa_v[0, i, pl.ds(16 * q, 16)] = v
        return 0

    lax.fori_loop(0, _SB, rowfn, 0)
    pltpu.sync_copy(pa_v.at[0], y_hbm.at[pl.ds(r0, _SB)])


@jax.jit
def _midsc_call(p, degp):
    k = functools.partial(
        pl.kernel,
        mesh=_mesh,
        compiler_params=pltpu.CompilerParams(use_tc_tiling_on_sc=False),
        out_type=jax.ShapeDtypeStruct((_N_PAD, _C), jnp.float32),
        scratch_types=[
            pltpu.VMEM((2, _SB, _C), jnp.float32),
            pltpu.VMEM((2, _SB, _DEG_W), jnp.float32),
        ],
    )(_midsc_body)
    return k(p, degp)


_BLK = 256


def _prep_body(feat_ref, w_ref, degp_ref, y0_ref):
    deg = degp_ref[0, :, 0] + degp_ref[1, :, 0]
    norm = lax.rsqrt(jnp.maximum(deg, 1.0))
    acc = jnp.dot(feat_ref[...], w_ref[...],
                  preferred_element_type=jnp.float32)
    y0_ref[...] = acc * norm[:, None]


@jax.jit
def _prep_call(featp, W, degp):
    return pl.pallas_call(
        _prep_body,
        grid=(_N_PAD // _BLK,),
        in_specs=[
            pl.BlockSpec((_BLK, _D), lambda i: (i, 0)),
            pl.BlockSpec((_D, _C), lambda i: (0, 0)),
            pl.BlockSpec((_NSC, _BLK, _DEG_W), lambda i: (0, i, 0)),
        ],
        out_specs=pl.BlockSpec((_BLK, _C), lambda i: (i, 0)),
        out_shape=jax.ShapeDtypeStruct((_N_PAD, _C), jnp.float32),
    )(featp, W, degp)


def _mid_body(p_ref, degp_ref, y_ref):
    deg = jnp.maximum(degp_ref[0, :, 0] + degp_ref[1, :, 0], 1.0)
    y_ref[...] = (p_ref[0] + p_ref[1]) * (1.0 / deg)[:, None]


@jax.jit
def _mid_call(p, degp):
    return pl.pallas_call(
        _mid_body,
        grid=(_N_PAD // _BLK,),
        in_specs=[
            pl.BlockSpec((_NSC, _BLK, _C), lambda i: (0, i, 0)),
            pl.BlockSpec((_NSC, _BLK, _DEG_W), lambda i: (0, i, 0)),
        ],
        out_specs=pl.BlockSpec((_BLK, _C), lambda i: (i, 0)),
        out_shape=jax.ShapeDtypeStruct((_N_PAD, _C), jnp.float32),
    )(p, degp)


def _fin_body(p_ref, degp_ref, out_ref, logits_ref):
    deg = jnp.maximum(degp_ref[0, :, 0] + degp_ref[1, :, 0], 1.0)
    norm = lax.rsqrt(deg)
    logits = (p_ref[0] + p_ref[1]) * norm[:, None]
    logits_ref[...] = logits
    m = jnp.max(logits, axis=1, keepdims=True)
    e = jnp.exp(logits - m)
    out_ref[...] = e / jnp.sum(e, axis=1, keepdims=True)


@jax.jit
def _fin_call(p, degp):
    return pl.pallas_call(
        _fin_body,
        grid=(_N_PAD // _BLK,),
        in_specs=[
            pl.BlockSpec((_NSC, _BLK, _C), lambda i: (0, i, 0)),
            pl.BlockSpec((_NSC, _BLK, _DEG_W), lambda i: (0, i, 0)),
        ],
        out_specs=[
            pl.BlockSpec((_BLK, _C), lambda i: (i, 0)),
            pl.BlockSpec((_BLK, _C), lambda i: (i, 0)),
        ],
        out_shape=[
            jax.ShapeDtypeStruct((_N_PAD, _C), jnp.float32),
            jax.ShapeDtypeStruct((_N_PAD, _C), jnp.float32),
        ],
    )(p, degp)


def kernel(features, edge_index, W):
    src = edge_index[0]
    dst = edge_index[1]
    pad_idx = jnp.full((_E_PAD - _E,), _N_PAD - 1, jnp.int32)
    srcp = jnp.concatenate([src, pad_idx]).reshape(_NW, _CH_PER_TILE, _CHUNK)
    dstp = jnp.concatenate([dst, pad_idx]).reshape(_NW, _CH_PER_TILE, _CHUNK)
    featp = jnp.pad(features, ((0, _N_PAD - _N), (0, 0)))
    zeros_pad = jnp.zeros((_N_PAD, _C), jnp.float32)
    ones_blk = jnp.ones((_CHUNK, _DEG_W), jnp.float32)
    zeros_deg = jnp.zeros((_N_PAD, _DEG_W), jnp.float32)

    degp = _deg_call(dstp, ones_blk, zeros_deg)
    y0 = _prep_call(featp, W, degp)
    p1 = _hop_call(y0, srcp, dstp, zeros_pad)
    y1 = _midsc_call(p1, degp)
    p2 = _hop_call(y1, srcp, dstp, zeros_pad)
    out_pad, logits_pad = _fin_call(p2, degp)
    return out_pad[:_N], logits_pad[:_N]
